# baseline (speedup 1.0000x reference)
import functools

import jax
import jax.numpy as jnp
from jax import lax
from jax.experimental import pallas as pl
import jax.experimental.pallas.tpu as pltpu
from jax.experimental.pallas import tpu_sc as plsc

GROUPS_K = 4
BETA_K = 0.25


def _score_body(z_ref, cb_ref, inds_ref, lsum_ref, lacc_ref,
                *, n_b, n_s):
    g = pl.program_id(0)
    b = pl.program_id(1)
    s = pl.program_id(2)

    @pl.when(jnp.logical_and(jnp.logical_and(g == 0, b == 0), s == 0))
    def _init_all():
        lacc_ref[0] = 0.0

    xT = z_ref[0]
    cb = cb_ref[0]
    cxx = jnp.sum(cb * cb, axis=1, keepdims=True)
    cb2 = cb * (-2.0)
    scoresT = jax.lax.dot_general(
        cb2, xT, (((1,), (0,)), ((), ())),
        preferred_element_type=jnp.float32)
    d2 = scoresT + cxx
    idx = jnp.argmin(d2, axis=0).astype(jnp.int32)
    inds_ref[0, 0, 0, :] = idx
    minv = jnp.min(d2, axis=0)
    sxx = jnp.sum(xT * xT, axis=0)
    lacc_ref[0] += jnp.sum(minv + sxx)

    last = jnp.logical_and(jnp.logical_and(g == GROUPS_K - 1, b == n_b - 1),
                           s == n_s - 1)

    @pl.when(last)
    def _fin_all():
        lsum_ref[:, :] = jnp.full((1, 1), lacc_ref[0], jnp.float32)


def _sc_body(cbt_ref, inds_ref, quant_ref, counts_ref,
             cbt_v, idx_v, row_v, cnt_v,
             *, n_e, dpg, hw, n_b, pairs_per_worker, halves):
    cid = lax.axis_index("c")
    sid = lax.axis_index("s")
    wid = sid * 2 + cid
    s_half = hw // halves
    n_vec = s_half // 16
    ones16 = jnp.full((16,), 1.0, jnp.float32)
    zeros16 = jnp.zeros((16,), jnp.float32)

    p0 = wid * pairs_per_worker
    g = p0 // n_b
    pltpu.sync_copy(cbt_ref.at[g], cbt_v)

    for j in range(pairs_per_worker):
        p = p0 + j
        b = p - g * n_b
        pltpu.sync_copy(inds_ref.at[p], idx_v)
        for k in range(n_e // 16):
            cnt_v[pl.ds(k * 16, 16)] = zeros16
        for h in range(halves):
            def gather_body(i, carry, h=h):
                vidx = idx_v[pl.ds(h * s_half + i * 16, 16)]
                plsc.addupdate_scatter(cnt_v, [vidx], ones16)
                for d in range(dpg):
                    row_v[d, pl.ds(i * 16, 16)] = plsc.load_gather(
                        cbt_v, [vidx + d * n_e])
                return carry
            lax.fori_loop(0, n_vec, gather_body, 0)
            pltpu.sync_copy(
                row_v,
                quant_ref.at[b, pl.ds(g * dpg, dpg), pl.ds(h * s_half, s_half)])
        pltpu.sync_copy(cnt_v, counts_ref.at[p])


def _perp_body(c1_ref, c2_ref, perps_ref, *, t):
    counts = jnp.sum(c1_ref[...], axis=1) + jnp.sum(c2_ref[...], axis=1)
    probs = counts * (1.0 / t)
    ent = -jnp.sum(probs * jnp.log(probs + 1e-10), axis=1, keepdims=True)
    perps_ref[:, :] = jnp.exp(ent)


def kernel(z, codebooks):
    z = z.astype(jnp.float32)
    B, C, H, W = z.shape
    HW = H * W
    G, N_E, DPG = codebooks.shape
    zr = z.reshape(B, C, HW)
    S = min(4096, HW)
    n_s = HW // S
    T = B * HW
    n_elems = T * DPG

    n_chunks = 2
    Bc = B // n_chunks

    def score_chunk(b0):
        body = functools.partial(_score_body, n_b=Bc, n_s=n_s)
        return pl.pallas_call(
            body,
            grid=(G, Bc, n_s),
            in_specs=[
                pl.BlockSpec((1, DPG, S), lambda g, b, s: (b + b0, g, s)),
                pl.BlockSpec((1, N_E, DPG), lambda g, b, s: (g, 0, 0)),
            ],
            out_specs=[
                pl.BlockSpec((1, 1, 1, S), lambda g, b, s: (g, b, 0, s)),
                pl.BlockSpec((1, 1), lambda g, b, s: (0, 0)),
            ],
            out_shape=[
                jax.ShapeDtypeStruct((G, Bc, 1, HW), jnp.int32),
                jax.ShapeDtypeStruct((1, 1), jnp.float32),
            ],
            scratch_shapes=[pltpu.SMEM((1,), jnp.float32)],
        )(zr, codebooks)

    cbt = jnp.transpose(codebooks, (0, 2, 1)).reshape(G, DPG * N_E)
    halves = 2
    mesh = plsc.VectorSubcoreMesh(core_axis_name="c", subcore_axis_name="s")
    sc = functools.partial(_sc_body, n_e=N_E, dpg=DPG, hw=HW, n_b=Bc,
                           pairs_per_worker=max(1, (G * Bc) // 32),
                           halves=halves)

    def lookup_chunk(inds_c):
        return pl.kernel(
            sc,
            out_type=[
                jax.ShapeDtypeStruct((Bc, C, HW), jnp.float32),
                jax.ShapeDtypeStruct((G * Bc, N_E), jnp.float32),
            ],
            mesh=mesh,
            scratch_types=[
                pltpu.VMEM((DPG * N_E,), jnp.float32),
                pltpu.VMEM((HW,), jnp.int32),
                pltpu.VMEM((DPG, HW // halves), jnp.float32),
                pltpu.VMEM((N_E,), jnp.float32),
            ],
            compiler_params=pltpu.CompilerParams(needs_layout_passes=False),
        )(cbt, inds_c.reshape(G * Bc, HW))

    inds_c, lsum_c, quant_c, counts_c = [], [], [], []
    for i in range(n_chunks):
        inds_i, lsum_i = score_chunk(i * Bc)
        q_i, c_i = lookup_chunk(inds_i)
        inds_c.append(inds_i)
        lsum_c.append(lsum_i)
        quant_c.append(q_i)
        counts_c.append(c_i)

    perp = functools.partial(_perp_body, t=float(T))
    perps = pl.pallas_call(
        perp,
        out_shape=jax.ShapeDtypeStruct((G, 1), jnp.float32),
    )(counts_c[0].reshape(G, Bc, N_E), counts_c[1].reshape(G, Bc, N_E))

    quantized = jnp.concatenate(quant_c, axis=0).reshape(B, C, H, W)
    inds = jnp.concatenate([ic.reshape(G, Bc, HW) for ic in inds_c], axis=1)
    loss = (1.0 + BETA_K) / n_elems * (lsum_c[0][0, 0] + lsum_c[1][0, 0])
    return (quantized, loss, perps[:, 0], inds)


# speedup vs baseline: 1.0789x; 1.0789x over previous
import functools

import jax
import jax.numpy as jnp
from jax import lax
from jax.experimental import pallas as pl
import jax.experimental.pallas.tpu as pltpu
from jax.experimental.pallas import tpu_sc as plsc

GROUPS_K = 4
BETA_K = 0.25


def _score_body(z_ref, cb_ref, inds_ref, loss_ref, lacc_ref,
                *, n_b, n_s, n_e, n_elems):
    g = pl.program_id(0)
    b = pl.program_id(1)
    s = pl.program_id(2)

    @pl.when(jnp.logical_and(jnp.logical_and(g == 0, b == 0), s == 0))
    def _init_all():
        lacc_ref[0] = 0.0

    xT = z_ref[0]
    cb = cb_ref[0]
    cxx = jnp.sum(cb * cb, axis=1, keepdims=True)
    cb2 = cb * (-2.0)
    scoresT = jax.lax.dot_general(
        cb2, xT, (((1,), (0,)), ((), ())),
        preferred_element_type=jnp.float32)
    d2 = scoresT + cxx
    idx = jnp.argmin(d2, axis=0).astype(jnp.int32)
    inds_ref[0, 0, 0, :] = idx
    minv = jnp.min(d2, axis=0)
    sxx = jnp.sum(xT * xT, axis=0)
    lacc_ref[0] += jnp.sum(minv + sxx)

    last = jnp.logical_and(jnp.logical_and(g == GROUPS_K - 1, b == n_b - 1),
                           s == n_s - 1)

    @pl.when(last)
    def _fin_all():
        total = (1.0 + BETA_K) * lacc_ref[0] / n_elems
        loss_ref[:, :] = jnp.full((1, 1), total, jnp.float32)


def _sc_body(cbt_ref, inds_ref, quant_ref, counts_ref,
             cbt_v, idx_v, row_v, cnt_v,
             *, n_e, dpg, hw, n_b, pairs_per_worker, halves):
    cid = lax.axis_index("c")
    sid = lax.axis_index("s")
    wid = sid * 2 + cid
    s_half = hw // halves
    n_vec = s_half // 16
    ones16 = jnp.full((16,), 1.0, jnp.float32)
    zeros16 = jnp.zeros((16,), jnp.float32)

    p0 = wid * pairs_per_worker
    g = p0 // n_b
    pltpu.sync_copy(cbt_ref.at[g], cbt_v)

    for j in range(pairs_per_worker):
        p = p0 + j
        b = p - g * n_b
        pltpu.sync_copy(inds_ref.at[p], idx_v)
        for k in range(n_e // 16):
            cnt_v[pl.ds(k * 16, 16)] = zeros16
        for h in range(halves):
            def gather_body(i, carry, h=h):
                vidx = idx_v[pl.ds(h * s_half + i * 16, 16)]
                plsc.addupdate_scatter(cnt_v, [vidx], ones16)
                for d in range(dpg):
                    row_v[d, pl.ds(i * 16, 16)] = plsc.load_gather(
                        cbt_v, [jnp.full((16,), d, jnp.int32), vidx])
                return carry
            lax.fori_loop(0, n_vec, gather_body, 0)
            pltpu.sync_copy(
                row_v,
                quant_ref.at[b, pl.ds(g * dpg, dpg), pl.ds(h * s_half, s_half)])
        pltpu.sync_copy(cnt_v, counts_ref.at[p])


def _perp_body(counts_ref, perps_ref, *, t):
    counts = jnp.sum(counts_ref[...], axis=1)
    probs = counts * (1.0 / t)
    ent = -jnp.sum(probs * jnp.log(probs + 1e-10), axis=1, keepdims=True)
    perps_ref[:, :] = jnp.exp(ent)


def kernel(z, codebooks):
    z = z.astype(jnp.float32)
    B, C, H, W = z.shape
    HW = H * W
    G, N_E, DPG = codebooks.shape
    zr = z.reshape(B, C, HW)
    S = min(4096, HW)
    n_s = HW // S
    T = B * HW
    n_elems = T * DPG

    score = functools.partial(_score_body, n_b=B, n_s=n_s, n_e=N_E,
                              n_elems=float(n_elems))
    inds4, loss = pl.pallas_call(
        score,
        grid=(G, B, n_s),
        in_specs=[
            pl.BlockSpec((1, DPG, S), lambda g, b, s: (b, g, s)),
            pl.BlockSpec((1, N_E, DPG), lambda g, b, s: (g, 0, 0)),
        ],
        out_specs=[
            pl.BlockSpec((1, 1, 1, S), lambda g, b, s: (g, b, 0, s)),
            pl.BlockSpec((1, 1), lambda g, b, s: (0, 0)),
        ],
        out_shape=[
            jax.ShapeDtypeStruct((G, B, 1, HW), jnp.int32),
            jax.ShapeDtypeStruct((1, 1), jnp.float32),
        ],
        scratch_shapes=[pltpu.SMEM((1,), jnp.float32)],
    )(zr, codebooks)

    cbt = jnp.transpose(codebooks, (0, 2, 1))
    inds_rows = inds4.reshape(G * B, HW)

    halves = 2
    sc = functools.partial(_sc_body, n_e=N_E, dpg=DPG, hw=HW, n_b=B,
                           pairs_per_worker=(G * B) // 32, halves=halves)
    mesh = plsc.VectorSubcoreMesh(core_axis_name="c", subcore_axis_name="s")
    quant3, counts_p = pl.kernel(
        sc,
        out_type=[
            jax.ShapeDtypeStruct((B, C, HW), jnp.float32),
            jax.ShapeDtypeStruct((G * B, N_E), jnp.float32),
        ],
        mesh=mesh,
        scratch_types=[
            pltpu.VMEM((DPG, N_E), jnp.float32),
            pltpu.VMEM((HW,), jnp.int32),
            pltpu.VMEM((DPG, HW // halves), jnp.float32),
            pltpu.VMEM((N_E,), jnp.float32),
        ],
        compiler_params=pltpu.CompilerParams(needs_layout_passes=False),
    )(cbt, inds_rows)

    perp = functools.partial(_perp_body, t=float(T))
    perps = pl.pallas_call(
        perp,
        out_shape=jax.ShapeDtypeStruct((G, 1), jnp.float32),
    )(counts_p.reshape(G, B, N_E))

    quantized = quant3.reshape(B, C, H, W)
    inds = inds4.reshape(G, B, HW)
    return (quantized, loss[0, 0], perps[:, 0], inds)
